# baseline (speedup 1.0000x reference)
import functools

import jax
import jax.numpy as jnp
from jax import lax
from jax.experimental import pallas as pl
from jax.experimental.pallas import tpu as pltpu
from jax.experimental.pallas import tpu_sc as plsc

_DIM = 32
_NC = 2
_NS = 16
_NW = _NC * _NS
_BC = 1024
_TBLK = 16384


def _transpose_body(tt_ref, out_ref):
    out_ref[...] = tt_ref[...].T


@functools.lru_cache(maxsize=None)
def _make_transpose(V: int):
    grid = (V + _TBLK - 1) // _TBLK
    return pl.pallas_call(
        _transpose_body,
        grid=(grid,),
        in_specs=[pl.BlockSpec((_DIM, _TBLK), lambda i: (0, i))],
        out_specs=pl.BlockSpec((_TBLK, _DIM), lambda i: (i, 0)),
        out_shape=jax.ShapeDtypeStruct((V, _DIM), jnp.float32),
    )


@functools.lru_cache(maxsize=None)
def _make_gather(F: int, B: int):
    n_chunks = B // _BC
    n_tasks = F * n_chunks
    tasks_per_w = n_tasks // _NW
    n_pairs = (tasks_per_w - 1) // 2
    nbb = _BC // 128
    mesh = plsc.VectorSubcoreMesh(core_axis_name="c", subcore_axis_name="s")

    @functools.partial(
        pl.kernel,
        mesh=mesh,
        out_type=jax.ShapeDtypeStruct((F, (_DIM // 8) * (B // 128) * 8, 128),
                                      jnp.float32),
        scratch_types=[
            pltpu.VMEM((_BC,), jnp.int32),
            pltpu.VMEM((_BC,), jnp.int32),
            pltpu.VMEM((_BC, _DIM), jnp.float32),
            pltpu.VMEM((_BC, _DIM), jnp.float32),
            pltpu.VMEM((_DIM // 8, nbb * 8, 129), jnp.float32),
            pltpu.SemaphoreType.DMA,
            pltpu.SemaphoreType.DMA,
        ],
        compiler_params=pltpu.CompilerParams(
            use_tc_tiling_on_sc=False, needs_layout_passes=False),
    )
    def _k(table_hbm, idx_hbm, out_hbm,
           idx_v0, idx_v1, rows_v0, rows_v1, tr_v, sem0, sem1):
        wid = lax.axis_index("s") * _NC + lax.axis_index("c")
        lane = lax.iota(jnp.int32, 16)
        idx_bufs = (idx_v0, idx_v1)
        rows_bufs = (rows_v0, rows_v1)
        sems = (sem0, sem1)

        def fc(k):
            t = wid + _NW * k
            return t // n_chunks, t % n_chunks

        def fire(k, b):
            f, c = fc(k)
            pltpu.sync_copy(idx_hbm.at[pl.ds(f * B + c * _BC, _BC)],
                            idx_bufs[b])
            pltpu.async_copy(table_hbm.at[idx_bufs[b]], rows_bufs[b], sems[b])

        def process(k, b):
            f, c = fc(k)
            pltpu.make_async_copy(table_hbm.at[idx_bufs[b]], rows_bufs[b],
                                  sems[b]).wait()

            lane7 = lane & 7
            off0 = (lane >> 3, (lane >> 3) + 2)

            @plsc.parallel_loop(0, _BC, 1, unroll=4)
            def tr_row(r):
                d1v = lane7 + ((r >> 7) << 3)
                d2v = lane * 0 + (r & 127)
                for h in range(2):
                    vals = rows_bufs[b][r, pl.ds(h * 16, 16)]
                    plsc.store_scatter(tr_v, [off0[h], d1v, d2v], vals)

            for dblk in range(_DIM // 8):
                pltpu.sync_copy(
                    tr_v.at[dblk, :, pl.ds(0, 128)],
                    out_hbm.at[f, pl.ds(dblk * (B // 16) + c * nbb * 8,
                                        nbb * 8), :],
                )

        fire(0, 0)

        def pair(g, carry):
            fire(2 * g + 1, 1)
            process(2 * g, 0)
            fire(2 * g + 2, 0)
            process(2 * g + 1, 1)
            return carry

        lax.fori_loop(0, n_pairs, pair, 0)
        process(tasks_per_w - 1, 0)

    return _k


def kernel(x, table):
    b, f = x.shape
    idx_f = jnp.reshape(x.T, (-1,))
    table_rm = _make_transpose(table.shape[0])(table.T)
    out_b = _make_gather(f, b)(table_rm, idx_f)
    out5 = jnp.reshape(out_b, (f, _DIM // 8, b // 128, 8, 128))
    return jnp.transpose(out5, (2, 4, 0, 1, 3)).reshape(b, f, _DIM)


# speedup vs baseline: 1.0818x; 1.0818x over previous
import functools

import jax
import jax.numpy as jnp
from jax import lax
from jax.experimental import pallas as pl
from jax.experimental.pallas import tpu as pltpu
from jax.experimental.pallas import tpu_sc as plsc

_DIM = 32
_NC = 2
_NS = 16
_NW = _NC * _NS
_BC = 1024


@functools.lru_cache(maxsize=None)
def _make_gather(F: int, B: int):
    n_chunks = B // _BC
    n_tasks = F * n_chunks
    tasks_per_w = n_tasks // _NW
    n_pairs = (tasks_per_w - 1) // 2
    nbb = _BC // 128
    mesh = plsc.VectorSubcoreMesh(core_axis_name="c", subcore_axis_name="s")

    @functools.partial(
        pl.kernel,
        mesh=mesh,
        out_type=jax.ShapeDtypeStruct((F, (_DIM // 8) * (B // 128) * 8, 128),
                                      jnp.float32),
        scratch_types=[
            pltpu.VMEM((_BC,), jnp.int32),
            pltpu.VMEM((_BC,), jnp.int32),
            pltpu.VMEM((_BC, _DIM), jnp.float32),
            pltpu.VMEM((_BC, _DIM), jnp.float32),
            pltpu.VMEM((_DIM // 8, nbb * 8, 129), jnp.float32),
            pltpu.SemaphoreType.DMA,
            pltpu.SemaphoreType.DMA,
        ],
        compiler_params=pltpu.CompilerParams(
            use_tc_tiling_on_sc=False, needs_layout_passes=False),
    )
    def _k(table_hbm, idx_hbm, out_hbm,
           idx_v0, idx_v1, rows_v0, rows_v1, tr_v, sem0, sem1):
        wid = lax.axis_index("s") * _NC + lax.axis_index("c")
        lane = lax.iota(jnp.int32, 16)
        idx_bufs = (idx_v0, idx_v1)
        rows_bufs = (rows_v0, rows_v1)
        sems = (sem0, sem1)

        def fc(k):
            t = wid + _NW * k
            return t // n_chunks, t % n_chunks

        def fire(k, b):
            f, c = fc(k)
            pltpu.sync_copy(idx_hbm.at[pl.ds(f * B + c * _BC, _BC)],
                            idx_bufs[b])
            pltpu.async_copy(table_hbm.at[idx_bufs[b]], rows_bufs[b], sems[b])

        def process(k, b):
            f, c = fc(k)
            pltpu.make_async_copy(table_hbm.at[idx_bufs[b]], rows_bufs[b],
                                  sems[b]).wait()

            lane7 = lane & 7
            off0 = (lane >> 3, (lane >> 3) + 2)

            @plsc.parallel_loop(0, _BC, 1, unroll=4)
            def tr_row(r):
                d1v = lane7 + ((r >> 7) << 3)
                d2v = lane * 0 + (r & 127)
                for h in range(2):
                    vals = rows_bufs[b][r, pl.ds(h * 16, 16)]
                    plsc.store_scatter(tr_v, [off0[h], d1v, d2v], vals)

            for dblk in range(_DIM // 8):
                pltpu.sync_copy(
                    tr_v.at[dblk, :, pl.ds(0, 128)],
                    out_hbm.at[f, pl.ds(dblk * (B // 16) + c * nbb * 8,
                                        nbb * 8), :],
                )

        fire(0, 0)

        def pair(g, carry):
            fire(2 * g + 1, 1)
            process(2 * g, 0)
            fire(2 * g + 2, 0)
            process(2 * g + 1, 1)
            return carry

        lax.fori_loop(0, n_pairs, pair, 0)
        process(tasks_per_w - 1, 0)

    return _k


def kernel(x, table):
    b, f = x.shape
    idx_f = jnp.reshape(x.T, (-1,))
    out_b = _make_gather(f, b)(table, idx_f)
    out5 = jnp.reshape(out_b, (f, _DIM // 8, b // 128, 8, 128))
    return jnp.transpose(out5, (2, 4, 0, 1, 3)).reshape(b, f, _DIM)
